# Initial kernel scaffold; baseline (speedup 1.0000x reference)
import jax
import jax.numpy as jnp
from jax.experimental import pallas as pl


def kernel(label_ids, table):
    raise NotImplementedError("write your pallas kernel here")


# speedup vs baseline: 1.0946x; 1.0946x over previous
import functools

import jax
import jax.numpy as jnp
from jax import lax
from jax.experimental import pallas as pl
from jax.experimental.pallas import tpu as pltpu
from jax.experimental.pallas import tpu_sc as plsc

_EMBED = 32
_B = 16384 * 50
_NC = 2
_NS = 16
_NW = _NC * _NS
_B_PER_W = _B // _NW
_CHUNK = 1024
_N_CHUNKS = _B_PER_W // _CHUNK

_mesh = plsc.VectorSubcoreMesh(core_axis_name="c", subcore_axis_name="s")


@functools.partial(
    pl.kernel,
    mesh=_mesh,
    out_type=jax.ShapeDtypeStruct((_B, _EMBED), jnp.float32),
    scratch_types=[
        pltpu.VMEM((_CHUNK,), jnp.int32),
        pltpu.VMEM((_CHUNK, _EMBED), jnp.float32),
        pltpu.SemaphoreType.DMA,
    ],
    compiler_params=pltpu.CompilerParams(use_tc_tiling_on_sc=False),
)
def _embed_gather(idx_hbm, table_hbm, out_hbm, idx_v, rows_v, sem):
    wid = lax.axis_index("s") * _NC + lax.axis_index("c")
    base = wid * _B_PER_W

    def body(i, carry):
        off = base + i * _CHUNK
        pltpu.sync_copy(idx_hbm.at[pl.ds(off, _CHUNK)], idx_v)
        pltpu.async_copy(table_hbm.at[idx_v], rows_v, sem).wait()
        pltpu.sync_copy(rows_v, out_hbm.at[pl.ds(off, _CHUNK)])
        return carry

    lax.fori_loop(0, _N_CHUNKS, body, 0)


def kernel(label_ids, table):
    idx = label_ids.reshape(-1)
    out = _embed_gather(idx, table)
    return out.reshape(label_ids.shape + (table.shape[1],))


# speedup vs baseline: 1.1078x; 1.0121x over previous
import functools

import jax
import jax.numpy as jnp
from jax import lax
from jax.experimental import pallas as pl
from jax.experimental.pallas import tpu as pltpu
from jax.experimental.pallas import tpu_sc as plsc

_EMBED = 32
_B = 16384 * 50
_NC = 2
_NS = 16
_NW = _NC * _NS
_B_PER_W = _B // _NW
_CHUNK = 1280
_N_CHUNKS = _B_PER_W // _CHUNK

_mesh = plsc.VectorSubcoreMesh(core_axis_name="c", subcore_axis_name="s")


@functools.partial(
    pl.kernel,
    mesh=_mesh,
    out_type=jax.ShapeDtypeStruct((_B, _EMBED), jnp.float32),
    scratch_types=[
        pltpu.VMEM((_CHUNK,), jnp.int32),
        pltpu.VMEM((_CHUNK,), jnp.int32),
        pltpu.VMEM((_CHUNK, _EMBED), jnp.float32),
        pltpu.VMEM((_CHUNK, _EMBED), jnp.float32),
        pltpu.SemaphoreType.DMA,
        pltpu.SemaphoreType.DMA,
        pltpu.SemaphoreType.DMA,
        pltpu.SemaphoreType.DMA,
    ],
    compiler_params=pltpu.CompilerParams(use_tc_tiling_on_sc=False),
)
def _embed_gather(idx_hbm, table_hbm, out_hbm,
                  idx_v0, idx_v1, rows_v0, rows_v1,
                  gsem0, gsem1, ssem0, ssem1):
    wid = lax.axis_index("s") * _NC + lax.axis_index("c")
    base = wid * _B_PER_W

    def idx_at(c):
        return idx_hbm.at[pl.ds(base + c * _CHUNK, _CHUNK)]

    def out_at(c):
        return out_hbm.at[pl.ds(base + c * _CHUNK, _CHUNK)]

    pltpu.sync_copy(idx_at(0), idx_v0)
    pltpu.async_copy(table_hbm.at[idx_v0], rows_v0, gsem0)

    def body(g, carry):
        c0 = 2 * g

        @pl.when(g > 0)
        def _():
            pltpu.make_async_copy(rows_v1, out_at(c0 - 1), ssem1).wait()

        pltpu.sync_copy(idx_at(c0 + 1), idx_v1)
        pltpu.async_copy(table_hbm.at[idx_v1], rows_v1, gsem1)
        pltpu.make_async_copy(table_hbm.at[idx_v0], rows_v0, gsem0).wait()
        pltpu.async_copy(rows_v0, out_at(c0), ssem0)

        pltpu.make_async_copy(rows_v0, out_at(c0), ssem0).wait()

        @pl.when(g < _N_CHUNKS // 2 - 1)
        def _():
            pltpu.sync_copy(idx_at(c0 + 2), idx_v0)
            pltpu.async_copy(table_hbm.at[idx_v0], rows_v0, gsem0)

        pltpu.make_async_copy(table_hbm.at[idx_v1], rows_v1, gsem1).wait()
        pltpu.async_copy(rows_v1, out_at(c0 + 1), ssem1)
        return carry

    lax.fori_loop(0, _N_CHUNKS // 2, body, 0)
    pltpu.make_async_copy(rows_v1, out_at(_N_CHUNKS - 1), ssem1).wait()


def kernel(label_ids, table):
    idx = label_ids.reshape(-1)
    out = _embed_gather(idx, table)
    return out.reshape(label_ids.shape + (table.shape[1],))


# speedup vs baseline: 1.8019x; 1.6265x over previous
import functools

import jax
import jax.numpy as jnp
from jax import lax
from jax.experimental import pallas as pl
from jax.experimental.pallas import tpu as pltpu
from jax.experimental.pallas import tpu_sc as plsc

_EMBED = 32
_BATCH = 16384
_HIST = 50
_NC = 2
_NS = 16
_NW = _NC * _NS
_ROWS_PER_W = _BATCH // _NW
_NB = 32
_N_CHUNKS = _ROWS_PER_W // _NB

_mesh = plsc.VectorSubcoreMesh(core_axis_name="c", subcore_axis_name="s")


@functools.partial(
    pl.kernel,
    mesh=_mesh,
    out_type=jax.ShapeDtypeStruct((_BATCH, _HIST, _EMBED), jnp.float32),
    scratch_types=[
        pltpu.VMEM((_NB, _HIST), jnp.int32),
        pltpu.VMEM((_NB, _HIST), jnp.int32),
        pltpu.VMEM((_NB, _HIST, _EMBED), jnp.float32),
        pltpu.VMEM((_NB, _HIST, _EMBED), jnp.float32),
        pltpu.SemaphoreType.DMA,
        pltpu.SemaphoreType.DMA,
        pltpu.SemaphoreType.DMA,
        pltpu.SemaphoreType.DMA,
    ],
    compiler_params=pltpu.CompilerParams(use_tc_tiling_on_sc=False),
)
def _embed_gather(idx_hbm, table_hbm, out_hbm,
                  idx_v0, idx_v1, rows_v0, rows_v1,
                  gsem0, gsem1, ssem0, ssem1):
    wid = lax.axis_index("s") * _NC + lax.axis_index("c")
    base = wid * _ROWS_PER_W

    def idx_at(c):
        return idx_hbm.at[pl.ds(base + c * _NB, _NB)]

    def out_at(c):
        return out_hbm.at[pl.ds(base + c * _NB, _NB)]

    def fire(idx_v, rows_v, sem):
        def j_body(j, carry):
            pltpu.async_copy(table_hbm.at[idx_v.at[j]], rows_v.at[j], sem)
            return carry
        lax.fori_loop(0, _NB, j_body, 0)

    def drain(idx_v, rows_v, sem):
        def j_body(j, carry):
            pltpu.make_async_copy(table_hbm.at[idx_v.at[j]], rows_v.at[j],
                                  sem).wait()
            return carry
        lax.fori_loop(0, _NB, j_body, 0)

    pltpu.sync_copy(idx_at(0), idx_v0)
    fire(idx_v0, rows_v0, gsem0)

    def body(g, carry):
        c0 = 2 * g

        @pl.when(g > 0)
        def _():
            pltpu.make_async_copy(rows_v1, out_at(c0 - 1), ssem1).wait()

        pltpu.sync_copy(idx_at(c0 + 1), idx_v1)
        fire(idx_v1, rows_v1, gsem1)
        drain(idx_v0, rows_v0, gsem0)
        pltpu.async_copy(rows_v0, out_at(c0), ssem0)

        pltpu.make_async_copy(rows_v0, out_at(c0), ssem0).wait()

        @pl.when(g < _N_CHUNKS // 2 - 1)
        def _():
            pltpu.sync_copy(idx_at(c0 + 2), idx_v0)
            fire(idx_v0, rows_v0, gsem0)

        drain(idx_v1, rows_v1, gsem1)
        pltpu.async_copy(rows_v1, out_at(c0 + 1), ssem1)
        return carry

    lax.fori_loop(0, _N_CHUNKS // 2, body, 0)
    pltpu.make_async_copy(rows_v1, out_at(_N_CHUNKS - 1), ssem1).wait()


def kernel(label_ids, table):
    return _embed_gather(label_ids, table)
